# Initial kernel scaffold; baseline (speedup 1.0000x reference)
import jax
import jax.numpy as jnp
from jax.experimental import pallas as pl


def kernel(input_ids, position_ids, attention_mask, word_emb, pos_emb):
    raise NotImplementedError("write your pallas kernel here")


# speedup vs baseline: 1.3091x; 1.3091x over previous
import functools

import jax
import jax.numpy as jnp
from jax import lax
from jax.experimental import pallas as pl
from jax.experimental.pallas import tpu as pltpu
from jax.experimental.pallas import tpu_sc as plsc

HIDDEN = 1024
NUM_CORES = 2
NUM_SUBCORES = 16
NW = NUM_CORES * NUM_SUBCORES
TOKENS = 4 * 2048
PER_W = TOKENS // NW
CHUNK = 32
NCHUNK = PER_W // CHUNK


def _embed_body(wids, pids, wtab, ptab, out, widx_v, pidx_v, buf_w, buf_p, sem_w, sem_p):
    wid = lax.axis_index("s") * NUM_CORES + lax.axis_index("c")
    pltpu.sync_copy(wids.at[wid], widx_v)
    pltpu.sync_copy(pids.at[wid], pidx_v)
    for j in range(NCHUNK):
        cw = pltpu.async_copy(wtab.at[widx_v.at[j]], buf_w, sem_w)
        cp = pltpu.async_copy(ptab.at[pidx_v.at[j]], buf_p, sem_p)
        cw.wait()
        cp.wait()

        @plsc.parallel_loop(0, CHUNK * HIDDEN // 16, unroll=8)
        def _add(t):
            r = t >> 6
            c = pl.multiple_of((t & 63) << 4, 16)
            plsc.addupdate(buf_w.at[r, pl.ds(c, 16)], buf_p[r, pl.ds(c, 16)])

        base = (wid * NCHUNK + j) * CHUNK
        pltpu.sync_copy(buf_w, out.at[pl.ds(base, CHUNK)])


@jax.jit
def kernel(input_ids, position_ids, attention_mask, word_emb, pos_emb):
    wids = input_ids.reshape(NW, NCHUNK, CHUNK).astype(jnp.int32)
    pids = position_ids.reshape(NW, NCHUNK, CHUNK).astype(jnp.int32)
    mesh = plsc.VectorSubcoreMesh(
        core_axis_name="c",
        subcore_axis_name="s",
        num_cores=NUM_CORES,
        num_subcores=NUM_SUBCORES,
    )
    out = pl.kernel(
        _embed_body,
        out_type=jax.ShapeDtypeStruct((TOKENS, HIDDEN), jnp.float32),
        mesh=mesh,
        scratch_types=[
            pltpu.VMEM((NCHUNK, CHUNK), jnp.int32),
            pltpu.VMEM((NCHUNK, CHUNK), jnp.int32),
            pltpu.VMEM((CHUNK, HIDDEN), jnp.float32),
            pltpu.VMEM((CHUNK, HIDDEN), jnp.float32),
            pltpu.SemaphoreType.DMA,
            pltpu.SemaphoreType.DMA,
        ],
    )(wids, pids, word_emb, pos_emb)
    b, s = input_ids.shape
    return out.reshape(b, s, HIDDEN), attention_mask


# speedup vs baseline: 1.6732x; 1.2781x over previous
import jax
import jax.numpy as jnp
from jax import lax
from jax.experimental import pallas as pl
from jax.experimental.pallas import tpu as pltpu
from jax.experimental.pallas import tpu_sc as plsc

HIDDEN = 1024
LANES = 16
NUM_CORES = 2
NUM_SUBCORES = 16
NW = NUM_CORES * NUM_SUBCORES
TOKENS = 4 * 2048
PER_W = TOKENS // NW
CHUNK = 16
NCHUNK = PER_W // CHUNK
CPH = HIDDEN // LANES


def _embed_body(wids, pids, wtab, ptab, out, widx_v, pidx_v, buf_w, buf_p,
                sem_w0, sem_w1, sem_p0, sem_p1, sem_o0, sem_o1):
    sem_w = (sem_w0, sem_w1)
    sem_p = (sem_p0, sem_p1)
    sem_o = (sem_o0, sem_o1)
    wid = lax.axis_index("s") * NUM_CORES + lax.axis_index("c")
    pltpu.sync_copy(wids.at[wid], widx_v)
    pltpu.sync_copy(pids.at[wid], pidx_v)

    gw = {}
    gp = {}
    oc = {}

    def issue(j):
        slot = j & 1
        gw[j] = pltpu.async_copy(wtab.at[widx_v.at[j]], buf_w.at[slot], sem_w[slot])
        gp[j] = pltpu.async_copy(ptab.at[pidx_v.at[j]], buf_p.at[slot], sem_p[slot])

    issue(0)
    for j in range(NCHUNK):
        slot = j & 1
        if j + 1 < NCHUNK:
            if j >= 1:
                oc[j - 1].wait()
            issue(j + 1)
        gw[j].wait()
        gp[j].wait()

        @plsc.parallel_loop(0, CHUNK * CPH, unroll=8)
        def _add(t):
            r = t >> 6
            c = pl.multiple_of((t & (CPH - 1)) << 4, LANES)
            plsc.addupdate(buf_w.at[slot, r, pl.ds(c, LANES)],
                           buf_p[slot, r, pl.ds(c, LANES)])

        base = (wid * NCHUNK + j) * CHUNK
        oc[j] = pltpu.async_copy(buf_w.at[slot], out.at[pl.ds(base, CHUNK)],
                                 sem_o[slot])
    oc[NCHUNK - 2].wait()
    oc[NCHUNK - 1].wait()


@jax.jit
def kernel(input_ids, position_ids, attention_mask, word_emb, pos_emb):
    wids = input_ids.reshape(NW, NCHUNK, CHUNK).astype(jnp.int32)
    pids = position_ids.reshape(NW, NCHUNK, CHUNK).astype(jnp.int32)
    mesh = plsc.VectorSubcoreMesh(
        core_axis_name="c",
        subcore_axis_name="s",
        num_cores=NUM_CORES,
        num_subcores=NUM_SUBCORES,
    )
    out = pl.kernel(
        _embed_body,
        out_type=jax.ShapeDtypeStruct((TOKENS, HIDDEN), jnp.float32),
        mesh=mesh,
        scratch_types=[
            pltpu.VMEM((NCHUNK, CHUNK), jnp.int32),
            pltpu.VMEM((NCHUNK, CHUNK), jnp.int32),
            pltpu.VMEM((2, CHUNK, HIDDEN), jnp.float32),
            pltpu.VMEM((2, CHUNK, HIDDEN), jnp.float32),
            pltpu.SemaphoreType.DMA,
            pltpu.SemaphoreType.DMA,
            pltpu.SemaphoreType.DMA,
            pltpu.SemaphoreType.DMA,
            pltpu.SemaphoreType.DMA,
            pltpu.SemaphoreType.DMA,
        ],
    )(wids, pids, word_emb, pos_emb)
    b, s = input_ids.shape
    return out.reshape(b, s, HIDDEN), attention_mask
